# baseline (speedup 1.0000x reference)
import functools

import jax
import jax.numpy as jnp
from jax import lax
from jax.experimental import pallas as pl
from jax.experimental.pallas import tpu as pltpu
from jax.experimental.pallas import tpu_sc as plsc

N = 10000
E = 320000
D = 128
H = 128
OUT = 32

NC = 2
NS = 16
NW = NC * NS
EPW = E // NW
CHUNK = 128
NG = 5
G = 16
NCHUNK = NG * G
PAD = NCHUNK * CHUNK - EPW
NTRASH = 128
ACC_ROWS = N + NTRASH
RPT = 624
RPT_LAST = N - (NS - 1) * RPT


def _sc_agg(feat, src_r, dst_r):
    mesh = plsc.VectorSubcoreMesh(core_axis_name="c", subcore_axis_name="s")

    @functools.partial(
        pl.kernel,
        out_type=jax.ShapeDtypeStruct((NC, N, D), jnp.float32),
        mesh=mesh,
        scratch_types=[
            pltpu.VMEM((G, CHUNK), jnp.int32),
            pltpu.VMEM((G, CHUNK), jnp.int32),
            pltpu.VMEM((CHUNK, D), jnp.float32),
            pltpu.VMEM((CHUNK, D), jnp.float32),
            pltpu.VMEM_SHARED((ACC_ROWS, D), jnp.float32),
            pltpu.SemaphoreType.DMA,
            pltpu.SemaphoreType.DMA,
        ],
    )
    def agg(feat_hbm, src_hbm, dst_hbm, out_hbm, src_grp, dst_grp, rows0, rows1,
            acc, semg0, semg1):
        c = lax.axis_index("c")
        s = lax.axis_index("s")
        wid = s * NC + c
        r0 = pl.multiple_of(s * RPT, 8)
        @pl.when(s < NS - 1)
        def _():
            pltpu.sync_copy(feat_hbm.at[pl.ds(r0, RPT)], acc.at[pl.ds(r0, RPT)])

        @pl.when(s == NS - 1)
        def _():
            pltpu.sync_copy(feat_hbm.at[pl.ds((NS - 1) * RPT, RPT_LAST)],
                            acc.at[pl.ds((NS - 1) * RPT, RPT_LAST)])

        plsc.subcore_barrier()

        def wait_g(rows, sem):
            pltpu.make_async_copy(feat_hbm.at[src_grp.at[0]], rows, sem).wait()

        def group_body(g, carry):
            pltpu.sync_copy(src_hbm.at[wid, g], src_grp)
            pltpu.sync_copy(dst_hbm.at[wid, g], dst_grp)
            pltpu.async_copy(feat_hbm.at[src_grp.at[0]], rows0, semg0)

            def pair(i, carry2):
                j0 = 2 * i
                pltpu.async_copy(feat_hbm.at[src_grp.at[j0 + 1]], rows1, semg1)
                wait_g(rows0, semg0)
                pltpu.sync_copy(rows0, acc.at[dst_grp.at[j0]], add=True)
                pltpu.async_copy(feat_hbm.at[src_grp.at[j0 + 2]], rows0, semg0)
                wait_g(rows1, semg1)
                pltpu.sync_copy(rows1, acc.at[dst_grp.at[j0 + 1]], add=True)
                return carry2

            lax.fori_loop(0, G // 2 - 1, pair, 0)
            pltpu.async_copy(feat_hbm.at[src_grp.at[G - 1]], rows1, semg1)
            wait_g(rows0, semg0)
            pltpu.sync_copy(rows0, acc.at[dst_grp.at[G - 2]], add=True)
            wait_g(rows1, semg1)
            pltpu.sync_copy(rows1, acc.at[dst_grp.at[G - 1]], add=True)
            return carry

        lax.fori_loop(0, NG, group_body, 0)
        plsc.subcore_barrier()

        @pl.when(s < NS - 1)
        def _():
            pltpu.sync_copy(acc.at[pl.ds(r0, RPT)], out_hbm.at[c, pl.ds(r0, RPT)])

        @pl.when(s == NS - 1)
        def _():
            pltpu.sync_copy(acc.at[pl.ds((NS - 1) * RPT, RPT_LAST)],
                            out_hbm.at[c, pl.ds((NS - 1) * RPT, RPT_LAST)])

    return agg(feat, src_r, dst_r)


def _ln(h, g, b):
    m = jnp.mean(h, axis=-1, keepdims=True)
    v = jnp.mean((h - m) * (h - m), axis=-1, keepdims=True)
    return (h - m) * lax.rsqrt(v + 1e-5) * g + b


NB = 10
BR = N // NB


def _mlp1_body(eps_ref, x_ref, p0_ref, p1_ref, w1_ref, b1_ref, g1_ref,
               be1_ref, w2_ref, b2_ref, ng_ref, nb_ref, o_ref):
    eps = eps_ref[0]
    h = p0_ref[...] + p1_ref[...] + (eps - 1.0) * x_ref[...]
    h = jnp.dot(h, w1_ref[...], preferred_element_type=jnp.float32) + b1_ref[...]
    h = jnp.maximum(_ln(h, g1_ref[...], be1_ref[...]), 0.0)
    h = jnp.dot(h, w2_ref[...], preferred_element_type=jnp.float32) + b2_ref[...]
    o_ref[...] = jnp.maximum(_ln(h, ng_ref[...], nb_ref[...]), 0.0)


def _mlp2_body(eps_ref, h_ref, q0_ref, q1_ref, w1_ref, b1_ref, g1_ref,
               be1_ref, w2_ref, b2_ref, ng_ref, nb_ref,
               rw1_ref, rb1_ref, rg_ref, rbe_ref, rw2_ref, rb2_ref, o_ref):
    eps = eps_ref[0]
    h = q0_ref[...] + q1_ref[...] + (eps - 1.0) * h_ref[...]
    h = jnp.dot(h, w1_ref[...], preferred_element_type=jnp.float32) + b1_ref[...]
    h = jnp.maximum(_ln(h, g1_ref[...], be1_ref[...]), 0.0)
    h = jnp.dot(h, w2_ref[...], preferred_element_type=jnp.float32) + b2_ref[...]
    h = jnp.maximum(_ln(h, ng_ref[...], nb_ref[...]), 0.0)
    o = jnp.dot(h, rw1_ref[...], preferred_element_type=jnp.float32) + rb1_ref[...]
    o = jnp.maximum(_ln(o, rg_ref[...], rbe_ref[...]), 0.0)
    o_ref[...] = jnp.dot(o, rw2_ref[...], preferred_element_type=jnp.float32) + rb2_ref[...]


def _row_spec(cols):
    return pl.BlockSpec((BR, cols), lambda i: (i, 0))


def _full_spec(r, c):
    return pl.BlockSpec((r, c), lambda i: (0, 0))


def _smem_spec():
    return pl.BlockSpec(memory_space=pltpu.SMEM)


def _mlp1(eps, x, p0, p1, w1, b1, g1, be1, w2, b2, ng, nb):
    return pl.pallas_call(
        _mlp1_body,
        grid=(NB,),
        in_specs=[
            _smem_spec(), _row_spec(D), _row_spec(D), _row_spec(D),
            _full_spec(D, H), _full_spec(1, H), _full_spec(1, H),
            _full_spec(1, H), _full_spec(H, H), _full_spec(1, H),
            _full_spec(1, H), _full_spec(1, H),
        ],
        out_specs=_row_spec(H),
        out_shape=jax.ShapeDtypeStruct((N, H), jnp.float32),
    )(eps.reshape(1), x, p0, p1, w1, b1.reshape(1, H), g1.reshape(1, H),
      be1.reshape(1, H), w2, b2.reshape(1, H), ng.reshape(1, H), nb.reshape(1, H))


def _mlp2(eps, h, q0, q1, w1, b1, g1, be1, w2, b2, ng, nb,
          rw1, rb1, rg, rbe, rw2, rb2):
    return pl.pallas_call(
        _mlp2_body,
        grid=(NB,),
        in_specs=[
            _smem_spec(), _row_spec(H), _row_spec(H), _row_spec(H),
            _full_spec(H, H), _full_spec(1, H), _full_spec(1, H),
            _full_spec(1, H), _full_spec(H, H), _full_spec(1, H),
            _full_spec(1, H), _full_spec(1, H),
            _full_spec(H, OUT), _full_spec(1, OUT), _full_spec(1, OUT),
            _full_spec(1, OUT), _full_spec(OUT, OUT), _full_spec(1, OUT),
        ],
        out_specs=_row_spec(OUT),
        out_shape=jax.ShapeDtypeStruct((N, OUT), jnp.float32),
    )(eps.reshape(1), h, q0, q1, w1, b1.reshape(1, H), g1.reshape(1, H),
      be1.reshape(1, H), w2, b2.reshape(1, H), ng.reshape(1, H), nb.reshape(1, H),
      rw1, rb1.reshape(1, OUT), rg.reshape(1, OUT), rbe.reshape(1, OUT),
      rw2, rb2.reshape(1, OUT))


def kernel(x, edge_index, edge_weight, eps1, m1_W1, m1_b1, m1_g1, m1_be1,
           m1_W2, m1_b2, n1_g, n1_b, eps2, m2_W1, m2_b1, m2_g1, m2_be1,
           m2_W2, m2_b2, n2_g, n2_b, r_W1, r_b1, r_g, r_be, r_W2, r_b2):
    ei_w = edge_index.reshape(2, NW, EPW)
    src_r = jnp.concatenate(
        [ei_w[0], jnp.zeros((NW, PAD), jnp.int32)], axis=1
    ).reshape(NW, NG, G, CHUNK)
    dst_r = jnp.concatenate(
        [ei_w[1], jnp.broadcast_to(N + jnp.arange(PAD, dtype=jnp.int32) % NTRASH,
                                   (NW, PAD))], axis=1
    ).reshape(NW, NG, G, CHUNK)
    p = _sc_agg(x, src_r, dst_r)
    h = _mlp1(eps1, x, p[0], p[1], m1_W1, m1_b1, m1_g1, m1_be1,
              m1_W2, m1_b2, n1_g, n1_b)
    q = _sc_agg(h, src_r, dst_r)
    return _mlp2(eps2, h, q[0], q[1], m2_W1, m2_b1, m2_g1, m2_be1,
                 m2_W2, m2_b2, n2_g, n2_b,
                 r_W1, r_b1, r_g, r_be, r_W2, r_b2)


# speedup vs baseline: 2.5743x; 2.5743x over previous
import functools

import jax
import jax.numpy as jnp
from jax import lax
from jax.experimental import pallas as pl
from jax.experimental.pallas import tpu as pltpu
from jax.experimental.pallas import tpu_sc as plsc

N = 10000
E = 320000
D = 128
H = 128
OUT = 32

NC = 2
NS = 16
NW = NC * NS
EPW = E // NW
CHUNK = 80
NCHUNK = EPW // CHUNK
NG = 5
G = NCHUNK // NG
ACC_ROWS = N
RPT = 624
RPT_LAST = N - (NS - 1) * RPT


def _sc_agg(feat, src_r, dst_r):
    mesh = plsc.VectorSubcoreMesh(core_axis_name="c", subcore_axis_name="s")

    @functools.partial(
        pl.kernel,
        out_type=jax.ShapeDtypeStruct((NC, N, D), jnp.float32),
        mesh=mesh,
        scratch_types=[
            pltpu.VMEM((G, CHUNK), jnp.int32),
            pltpu.VMEM((G, CHUNK), jnp.int32),
            pltpu.VMEM((CHUNK, D), jnp.float32),
            pltpu.VMEM((CHUNK, D), jnp.float32),
            pltpu.VMEM_SHARED((ACC_ROWS, D), jnp.float32),
            pltpu.SemaphoreType.DMA,
            pltpu.SemaphoreType.DMA,
        ],
    )
    def agg(feat_hbm, src_hbm, dst_hbm, out_hbm, src_grp, dst_grp, rows0, rows1,
            acc, semg0, semg1):
        c = lax.axis_index("c")
        s = lax.axis_index("s")
        wid = s * NC + c
        r0 = pl.multiple_of(s * RPT, 8)
        @pl.when(s < NS - 1)
        def _():
            pltpu.sync_copy(feat_hbm.at[pl.ds(r0, RPT)], acc.at[pl.ds(r0, RPT)])

        @pl.when(s == NS - 1)
        def _():
            pltpu.sync_copy(feat_hbm.at[pl.ds((NS - 1) * RPT, RPT_LAST)],
                            acc.at[pl.ds((NS - 1) * RPT, RPT_LAST)])

        plsc.subcore_barrier()

        def wait_g(rows, sem):
            pltpu.make_async_copy(feat_hbm.at[src_grp.at[0]], rows, sem).wait()

        def group_body(g, carry):
            pltpu.sync_copy(src_hbm.at[wid, g], src_grp)
            pltpu.sync_copy(dst_hbm.at[wid, g], dst_grp)
            pltpu.async_copy(feat_hbm.at[src_grp.at[0]], rows0, semg0)

            def pair(i, carry2):
                j0 = 2 * i
                pltpu.async_copy(feat_hbm.at[src_grp.at[j0 + 1]], rows1, semg1)
                wait_g(rows0, semg0)
                pltpu.sync_copy(rows0, acc.at[dst_grp.at[j0]], add=True)
                pltpu.async_copy(feat_hbm.at[src_grp.at[j0 + 2]], rows0, semg0)
                wait_g(rows1, semg1)
                pltpu.sync_copy(rows1, acc.at[dst_grp.at[j0 + 1]], add=True)
                return carry2

            lax.fori_loop(0, (G - 1) // 2, pair, 0)
            wait_g(rows0, semg0)
            pltpu.sync_copy(rows0, acc.at[dst_grp.at[G - 1]], add=True)
            return carry

        lax.fori_loop(0, NG, group_body, 0)
        plsc.subcore_barrier()

        @pl.when(s < NS - 1)
        def _():
            pltpu.sync_copy(acc.at[pl.ds(r0, RPT)], out_hbm.at[c, pl.ds(r0, RPT)])

        @pl.when(s == NS - 1)
        def _():
            pltpu.sync_copy(acc.at[pl.ds((NS - 1) * RPT, RPT_LAST)],
                            out_hbm.at[c, pl.ds((NS - 1) * RPT, RPT_LAST)])

    return agg(feat, src_r, dst_r)


def _ln(h, g, b):
    m = jnp.mean(h, axis=-1, keepdims=True)
    v = jnp.mean((h - m) * (h - m), axis=-1, keepdims=True)
    return (h - m) * lax.rsqrt(v + 1e-5) * g + b


NB = 10
BR = N // NB


def _mlp1_body(eps_ref, x_ref, p0_ref, p1_ref, w1_ref, b1_ref, g1_ref,
               be1_ref, w2_ref, b2_ref, ng_ref, nb_ref, o_ref):
    eps = eps_ref[0]
    h = p0_ref[...] + p1_ref[...] + (eps - 1.0) * x_ref[...]
    h = jnp.dot(h, w1_ref[...], preferred_element_type=jnp.float32) + b1_ref[...]
    h = jnp.maximum(_ln(h, g1_ref[...], be1_ref[...]), 0.0)
    h = jnp.dot(h, w2_ref[...], preferred_element_type=jnp.float32) + b2_ref[...]
    o_ref[...] = jnp.maximum(_ln(h, ng_ref[...], nb_ref[...]), 0.0)


def _mlp2_body(eps_ref, h_ref, q0_ref, q1_ref, w1_ref, b1_ref, g1_ref,
               be1_ref, w2_ref, b2_ref, ng_ref, nb_ref,
               rw1_ref, rb1_ref, rg_ref, rbe_ref, rw2_ref, rb2_ref, o_ref):
    eps = eps_ref[0]
    h = q0_ref[...] + q1_ref[...] + (eps - 1.0) * h_ref[...]
    h = jnp.dot(h, w1_ref[...], preferred_element_type=jnp.float32) + b1_ref[...]
    h = jnp.maximum(_ln(h, g1_ref[...], be1_ref[...]), 0.0)
    h = jnp.dot(h, w2_ref[...], preferred_element_type=jnp.float32) + b2_ref[...]
    h = jnp.maximum(_ln(h, ng_ref[...], nb_ref[...]), 0.0)
    o = jnp.dot(h, rw1_ref[...], preferred_element_type=jnp.float32) + rb1_ref[...]
    o = jnp.maximum(_ln(o, rg_ref[...], rbe_ref[...]), 0.0)
    o_ref[...] = jnp.dot(o, rw2_ref[...], preferred_element_type=jnp.float32) + rb2_ref[...]


def _row_spec(cols):
    return pl.BlockSpec((BR, cols), lambda i: (i, 0))


def _full_spec(r, c):
    return pl.BlockSpec((r, c), lambda i: (0, 0))


def _smem_spec():
    return pl.BlockSpec(memory_space=pltpu.SMEM)


def _mlp1(eps, x, p0, p1, w1, b1, g1, be1, w2, b2, ng, nb):
    return pl.pallas_call(
        _mlp1_body,
        grid=(NB,),
        in_specs=[
            _smem_spec(), _row_spec(D), _row_spec(D), _row_spec(D),
            _full_spec(D, H), _full_spec(1, H), _full_spec(1, H),
            _full_spec(1, H), _full_spec(H, H), _full_spec(1, H),
            _full_spec(1, H), _full_spec(1, H),
        ],
        out_specs=_row_spec(H),
        out_shape=jax.ShapeDtypeStruct((N, H), jnp.float32),
    )(eps.reshape(1), x, p0, p1, w1, b1.reshape(1, H), g1.reshape(1, H),
      be1.reshape(1, H), w2, b2.reshape(1, H), ng.reshape(1, H), nb.reshape(1, H))


def _mlp2(eps, h, q0, q1, w1, b1, g1, be1, w2, b2, ng, nb,
          rw1, rb1, rg, rbe, rw2, rb2):
    return pl.pallas_call(
        _mlp2_body,
        grid=(NB,),
        in_specs=[
            _smem_spec(), _row_spec(H), _row_spec(H), _row_spec(H),
            _full_spec(H, H), _full_spec(1, H), _full_spec(1, H),
            _full_spec(1, H), _full_spec(H, H), _full_spec(1, H),
            _full_spec(1, H), _full_spec(1, H),
            _full_spec(H, OUT), _full_spec(1, OUT), _full_spec(1, OUT),
            _full_spec(1, OUT), _full_spec(OUT, OUT), _full_spec(1, OUT),
        ],
        out_specs=_row_spec(OUT),
        out_shape=jax.ShapeDtypeStruct((N, OUT), jnp.float32),
    )(eps.reshape(1), h, q0, q1, w1, b1.reshape(1, H), g1.reshape(1, H),
      be1.reshape(1, H), w2, b2.reshape(1, H), ng.reshape(1, H), nb.reshape(1, H),
      rw1, rb1.reshape(1, OUT), rg.reshape(1, OUT), rbe.reshape(1, OUT),
      rw2, rb2.reshape(1, OUT))


def kernel(x, edge_index, edge_weight, eps1, m1_W1, m1_b1, m1_g1, m1_be1,
           m1_W2, m1_b2, n1_g, n1_b, eps2, m2_W1, m2_b1, m2_g1, m2_be1,
           m2_W2, m2_b2, n2_g, n2_b, r_W1, r_b1, r_g, r_be, r_W2, r_b2):
    ei_r = edge_index.reshape(2, NW, NG, G, CHUNK)
    src_r, dst_r = ei_r[0], ei_r[1]
    p = _sc_agg(x, src_r, dst_r)
    h = _mlp1(eps1, x, p[0], p[1], m1_W1, m1_b1, m1_g1, m1_be1,
              m1_W2, m1_b2, n1_g, n1_b)
    q = _sc_agg(h, src_r, dst_r)
    return _mlp2(eps2, h, q[0], q[1], m2_W1, m2_b1, m2_g1, m2_be1,
                 m2_W2, m2_b2, n2_g, n2_b,
                 r_W1, r_b1, r_g, r_be, r_W2, r_b2)


# speedup vs baseline: 2.7263x; 1.0590x over previous
import functools

import jax
import jax.numpy as jnp
from jax import lax
from jax.experimental import pallas as pl
from jax.experimental.pallas import tpu as pltpu
from jax.experimental.pallas import tpu_sc as plsc

N = 10000
E = 320000
D = 128
H = 128
OUT = 32

NC = 2
NS = 16
NW = NC * NS
EPW = E // NW
CHUNK = 80
NCHUNK = EPW // CHUNK
NG = 5
G = NCHUNK // NG
ACC_ROWS = N
RPT = 624
RPT_LAST = N - (NS - 1) * RPT


def _sc_agg(feat, ei_resh):
    mesh = plsc.VectorSubcoreMesh(core_axis_name="c", subcore_axis_name="s")

    @functools.partial(
        pl.kernel,
        out_type=jax.ShapeDtypeStruct((NC, N, D), jnp.float32),
        mesh=mesh,
        scratch_types=[
            pltpu.VMEM((G, CHUNK), jnp.int32),
            pltpu.VMEM((G, CHUNK), jnp.int32),
            pltpu.VMEM((CHUNK, D), jnp.float32),
            pltpu.VMEM((CHUNK, D), jnp.float32),
            pltpu.VMEM_SHARED((ACC_ROWS, D), jnp.float32),
            pltpu.SemaphoreType.DMA,
            pltpu.SemaphoreType.DMA,
        ],
    )
    def agg(feat_hbm, ei_hbm, out_hbm, src_grp, dst_grp, rows0, rows1,
            acc, semg0, semg1):
        c = lax.axis_index("c")
        s = lax.axis_index("s")
        wid = s * NC + c
        r0 = pl.multiple_of(s * RPT, 8)
        @pl.when(s < NS - 1)
        def _():
            pltpu.sync_copy(feat_hbm.at[pl.ds(r0, RPT)], acc.at[pl.ds(r0, RPT)])

        @pl.when(s == NS - 1)
        def _():
            pltpu.sync_copy(feat_hbm.at[pl.ds((NS - 1) * RPT, RPT_LAST)],
                            acc.at[pl.ds((NS - 1) * RPT, RPT_LAST)])

        plsc.subcore_barrier()

        def wait_g(rows, sem):
            pltpu.make_async_copy(feat_hbm.at[src_grp.at[0]], rows, sem).wait()

        def group_body(g, carry):
            pltpu.sync_copy(ei_hbm.at[0, wid, g], src_grp)
            pltpu.sync_copy(ei_hbm.at[1, wid, g], dst_grp)
            pltpu.async_copy(feat_hbm.at[src_grp.at[0]], rows0, semg0)

            def pair(i, carry2):
                j0 = 2 * i
                pltpu.async_copy(feat_hbm.at[src_grp.at[j0 + 1]], rows1, semg1)
                wait_g(rows0, semg0)
                pltpu.sync_copy(rows0, acc.at[dst_grp.at[j0]], add=True)
                pltpu.async_copy(feat_hbm.at[src_grp.at[j0 + 2]], rows0, semg0)
                wait_g(rows1, semg1)
                pltpu.sync_copy(rows1, acc.at[dst_grp.at[j0 + 1]], add=True)
                return carry2

            lax.fori_loop(0, (G - 1) // 2, pair, 0)
            wait_g(rows0, semg0)
            pltpu.sync_copy(rows0, acc.at[dst_grp.at[G - 1]], add=True)
            return carry

        lax.fori_loop(0, NG, group_body, 0)
        plsc.subcore_barrier()

        @pl.when(s < NS - 1)
        def _():
            pltpu.sync_copy(acc.at[pl.ds(r0, RPT)], out_hbm.at[c, pl.ds(r0, RPT)])

        @pl.when(s == NS - 1)
        def _():
            pltpu.sync_copy(acc.at[pl.ds((NS - 1) * RPT, RPT_LAST)],
                            out_hbm.at[c, pl.ds((NS - 1) * RPT, RPT_LAST)])

    return agg(feat, ei_resh)


def _ln(h, g, b):
    m = jnp.mean(h, axis=-1, keepdims=True)
    v = jnp.mean((h - m) * (h - m), axis=-1, keepdims=True)
    return (h - m) * lax.rsqrt(v + 1e-5) * g + b


NB = 10
BR = N // NB


def _mlp1_body(eps_ref, x_ref, p0_ref, p1_ref, w1_ref, b1_ref, g1_ref,
               be1_ref, w2_ref, b2_ref, ng_ref, nb_ref, o_ref):
    eps = eps_ref[0]
    h = p0_ref[...] + p1_ref[...] + (eps - 1.0) * x_ref[...]
    h = jnp.dot(h, w1_ref[...], preferred_element_type=jnp.float32) + b1_ref[...]
    h = jnp.maximum(_ln(h, g1_ref[...], be1_ref[...]), 0.0)
    h = jnp.dot(h, w2_ref[...], preferred_element_type=jnp.float32) + b2_ref[...]
    o_ref[...] = jnp.maximum(_ln(h, ng_ref[...], nb_ref[...]), 0.0)


def _mlp2_body(eps_ref, h_ref, q0_ref, q1_ref, w1_ref, b1_ref, g1_ref,
               be1_ref, w2_ref, b2_ref, ng_ref, nb_ref,
               rw1_ref, rb1_ref, rg_ref, rbe_ref, rw2_ref, rb2_ref, o_ref):
    eps = eps_ref[0]
    h = q0_ref[...] + q1_ref[...] + (eps - 1.0) * h_ref[...]
    h = jnp.dot(h, w1_ref[...], preferred_element_type=jnp.float32) + b1_ref[...]
    h = jnp.maximum(_ln(h, g1_ref[...], be1_ref[...]), 0.0)
    h = jnp.dot(h, w2_ref[...], preferred_element_type=jnp.float32) + b2_ref[...]
    h = jnp.maximum(_ln(h, ng_ref[...], nb_ref[...]), 0.0)
    o = jnp.dot(h, rw1_ref[...], preferred_element_type=jnp.float32) + rb1_ref[...]
    o = jnp.maximum(_ln(o, rg_ref[...], rbe_ref[...]), 0.0)
    o_ref[...] = jnp.dot(o, rw2_ref[...], preferred_element_type=jnp.float32) + rb2_ref[...]


def _row_spec(cols):
    return pl.BlockSpec((BR, cols), lambda i: (i, 0))


def _full_spec(r, c):
    return pl.BlockSpec((r, c), lambda i: (0, 0))


def _smem_spec():
    return pl.BlockSpec(memory_space=pltpu.SMEM)


def _mlp1(eps, x, p0, p1, w1, b1, g1, be1, w2, b2, ng, nb):
    return pl.pallas_call(
        _mlp1_body,
        grid=(NB,),
        in_specs=[
            _smem_spec(), _row_spec(D), _row_spec(D), _row_spec(D),
            _full_spec(D, H), _full_spec(1, H), _full_spec(1, H),
            _full_spec(1, H), _full_spec(H, H), _full_spec(1, H),
            _full_spec(1, H), _full_spec(1, H),
        ],
        out_specs=_row_spec(H),
        out_shape=jax.ShapeDtypeStruct((N, H), jnp.float32),
    )(eps.reshape(1), x, p0, p1, w1, b1.reshape(1, H), g1.reshape(1, H),
      be1.reshape(1, H), w2, b2.reshape(1, H), ng.reshape(1, H), nb.reshape(1, H))


def _mlp2(eps, h, q0, q1, w1, b1, g1, be1, w2, b2, ng, nb,
          rw1, rb1, rg, rbe, rw2, rb2):
    return pl.pallas_call(
        _mlp2_body,
        grid=(NB,),
        in_specs=[
            _smem_spec(), _row_spec(H), _row_spec(H), _row_spec(H),
            _full_spec(H, H), _full_spec(1, H), _full_spec(1, H),
            _full_spec(1, H), _full_spec(H, H), _full_spec(1, H),
            _full_spec(1, H), _full_spec(1, H),
            _full_spec(H, OUT), _full_spec(1, OUT), _full_spec(1, OUT),
            _full_spec(1, OUT), _full_spec(OUT, OUT), _full_spec(1, OUT),
        ],
        out_specs=_row_spec(OUT),
        out_shape=jax.ShapeDtypeStruct((N, OUT), jnp.float32),
    )(eps.reshape(1), h, q0, q1, w1, b1.reshape(1, H), g1.reshape(1, H),
      be1.reshape(1, H), w2, b2.reshape(1, H), ng.reshape(1, H), nb.reshape(1, H),
      rw1, rb1.reshape(1, OUT), rg.reshape(1, OUT), rbe.reshape(1, OUT),
      rw2, rb2.reshape(1, OUT))


def kernel(x, edge_index, edge_weight, eps1, m1_W1, m1_b1, m1_g1, m1_be1,
           m1_W2, m1_b2, n1_g, n1_b, eps2, m2_W1, m2_b1, m2_g1, m2_be1,
           m2_W2, m2_b2, n2_g, n2_b, r_W1, r_b1, r_g, r_be, r_W2, r_b2):
    ei_resh = edge_index.reshape(2, NW, NG, G, CHUNK)
    p = _sc_agg(x, ei_resh)
    h = _mlp1(eps1, x, p[0], p[1], m1_W1, m1_b1, m1_g1, m1_be1,
              m1_W2, m1_b2, n1_g, n1_b)
    q = _sc_agg(h, ei_resh)
    return _mlp2(eps2, h, q[0], q[1], m2_W1, m2_b1, m2_g1, m2_be1,
                 m2_W2, m2_b2, n2_g, n2_b,
                 r_W1, r_b1, r_g, r_be, r_W2, r_b2)


# speedup vs baseline: 2.9123x; 1.0682x over previous
import functools

import jax
import jax.numpy as jnp
from jax import lax
from jax.experimental import pallas as pl
from jax.experimental.pallas import tpu as pltpu
from jax.experimental.pallas import tpu_sc as plsc

N = 10000
E = 320000
D = 128
H = 128
OUT = 32

NC = 2
NS = 16
NW = NC * NS
EPW = E // NW
CHUNK = 80
NCHUNK = EPW // CHUNK
NG = 5
G = NCHUNK // NG
ACC_ROWS = N
RPT = 624
RPT_LAST = N - (NS - 1) * RPT


def _sc_agg(feat, ei_resh):
    mesh = plsc.VectorSubcoreMesh(core_axis_name="c", subcore_axis_name="s")

    @functools.partial(
        pl.kernel,
        out_type=jax.ShapeDtypeStruct((NC, N, D), jnp.float32),
        mesh=mesh,
        scratch_types=[
            pltpu.VMEM((G, CHUNK), jnp.int32),
            pltpu.VMEM((G, CHUNK), jnp.int32),
            pltpu.VMEM((G, CHUNK), jnp.int32),
            pltpu.VMEM((G, CHUNK), jnp.int32),
            pltpu.VMEM((CHUNK, D), jnp.float32),
            pltpu.VMEM((CHUNK, D), jnp.float32),
            pltpu.VMEM_SHARED((ACC_ROWS, D), jnp.float32),
            pltpu.SemaphoreType.DMA,
            pltpu.SemaphoreType.DMA,
            pltpu.SemaphoreType.DMA,
        ],
    )
    def agg(feat_hbm, ei_hbm, out_hbm, src_a, dst_a, src_b, dst_b,
            rows0, rows1, acc, semg0, semg1, semi):
        c = lax.axis_index("c")
        s = lax.axis_index("s")
        wid = s * NC + c
        r0 = pl.multiple_of(s * RPT, 8)
        pltpu.async_copy(ei_hbm.at[0, wid, 0], src_a, semi)
        pltpu.async_copy(ei_hbm.at[1, wid, 0], dst_a, semi)
        @pl.when(s < NS - 1)
        def _():
            pltpu.sync_copy(feat_hbm.at[pl.ds(r0, RPT)], acc.at[pl.ds(r0, RPT)])

        @pl.when(s == NS - 1)
        def _():
            pltpu.sync_copy(feat_hbm.at[pl.ds((NS - 1) * RPT, RPT_LAST)],
                            acc.at[pl.ds((NS - 1) * RPT, RPT_LAST)])

        def wait_idx(buf):
            pltpu.make_async_copy(ei_hbm.at[0, wid, 0], buf, semi).wait()

        def wait_g(rows, sem):
            pltpu.make_async_copy(feat_hbm.at[src_a.at[0]], rows, sem).wait()

        wait_idx(src_a)
        wait_idx(dst_a)
        pltpu.async_copy(feat_hbm.at[src_a.at[0]], rows0, semg0)
        plsc.subcore_barrier()

        for g in range(NG):
            cs, cd = (src_a, dst_a) if g % 2 == 0 else (src_b, dst_b)
            ns_, nd = (src_b, dst_b) if g % 2 == 0 else (src_a, dst_a)
            ra, sa = (rows0, semg0) if g % 2 == 0 else (rows1, semg1)
            rb, sb = (rows1, semg1) if g % 2 == 0 else (rows0, semg0)
            if g + 1 < NG:
                pltpu.async_copy(ei_hbm.at[0, wid, g + 1], ns_, semi)
                pltpu.async_copy(ei_hbm.at[1, wid, g + 1], nd, semi)

            def pair(i, carry2, cs=cs, cd=cd, ra=ra, sa=sa, rb=rb, sb=sb):
                j0 = 2 * i
                pltpu.async_copy(feat_hbm.at[cs.at[j0 + 1]], rb, sb)
                wait_g(ra, sa)
                pltpu.sync_copy(ra, acc.at[cd.at[j0]], add=True)
                pltpu.async_copy(feat_hbm.at[cs.at[j0 + 2]], ra, sa)
                wait_g(rb, sb)
                pltpu.sync_copy(rb, acc.at[cd.at[j0 + 1]], add=True)
                return carry2

            lax.fori_loop(0, (G - 1) // 2, pair, 0)
            if g + 1 < NG:
                wait_idx(ns_)
                wait_idx(nd)
                pltpu.async_copy(feat_hbm.at[ns_.at[0]], rb, sb)
            wait_g(ra, sa)
            pltpu.sync_copy(ra, acc.at[cd.at[G - 1]], add=True)

        plsc.subcore_barrier()

        @pl.when(s < NS - 1)
        def _():
            pltpu.sync_copy(acc.at[pl.ds(r0, RPT)], out_hbm.at[c, pl.ds(r0, RPT)])

        @pl.when(s == NS - 1)
        def _():
            pltpu.sync_copy(acc.at[pl.ds((NS - 1) * RPT, RPT_LAST)],
                            out_hbm.at[c, pl.ds((NS - 1) * RPT, RPT_LAST)])

    return agg(feat, ei_resh)


def _ln(h, g, b):
    m = jnp.mean(h, axis=-1, keepdims=True)
    v = jnp.mean((h - m) * (h - m), axis=-1, keepdims=True)
    return (h - m) * lax.rsqrt(v + 1e-5) * g + b


NB = 10
BR = N // NB


def _mlp1_body(eps_ref, x_ref, p0_ref, p1_ref, w1_ref, b1_ref, g1_ref,
               be1_ref, w2_ref, b2_ref, ng_ref, nb_ref, o_ref):
    eps = eps_ref[0]
    h = p0_ref[...] + p1_ref[...] + (eps - 1.0) * x_ref[...]
    h = jnp.dot(h, w1_ref[...], preferred_element_type=jnp.float32) + b1_ref[...]
    h = jnp.maximum(_ln(h, g1_ref[...], be1_ref[...]), 0.0)
    h = jnp.dot(h, w2_ref[...], preferred_element_type=jnp.float32) + b2_ref[...]
    o_ref[...] = jnp.maximum(_ln(h, ng_ref[...], nb_ref[...]), 0.0)


def _mlp2_body(eps_ref, h_ref, q0_ref, q1_ref, w1_ref, b1_ref, g1_ref,
               be1_ref, w2_ref, b2_ref, ng_ref, nb_ref,
               rw1_ref, rb1_ref, rg_ref, rbe_ref, rw2_ref, rb2_ref, o_ref):
    eps = eps_ref[0]
    h = q0_ref[...] + q1_ref[...] + (eps - 1.0) * h_ref[...]
    h = jnp.dot(h, w1_ref[...], preferred_element_type=jnp.float32) + b1_ref[...]
    h = jnp.maximum(_ln(h, g1_ref[...], be1_ref[...]), 0.0)
    h = jnp.dot(h, w2_ref[...], preferred_element_type=jnp.float32) + b2_ref[...]
    h = jnp.maximum(_ln(h, ng_ref[...], nb_ref[...]), 0.0)
    o = jnp.dot(h, rw1_ref[...], preferred_element_type=jnp.float32) + rb1_ref[...]
    o = jnp.maximum(_ln(o, rg_ref[...], rbe_ref[...]), 0.0)
    o_ref[...] = jnp.dot(o, rw2_ref[...], preferred_element_type=jnp.float32) + rb2_ref[...]


def _row_spec(cols):
    return pl.BlockSpec((BR, cols), lambda i: (i, 0))


def _full_spec(r, c):
    return pl.BlockSpec((r, c), lambda i: (0, 0))


def _smem_spec():
    return pl.BlockSpec(memory_space=pltpu.SMEM)


def _mlp1(eps, x, p0, p1, w1, b1, g1, be1, w2, b2, ng, nb):
    return pl.pallas_call(
        _mlp1_body,
        grid=(NB,),
        in_specs=[
            _smem_spec(), _row_spec(D), _row_spec(D), _row_spec(D),
            _full_spec(D, H), _full_spec(1, H), _full_spec(1, H),
            _full_spec(1, H), _full_spec(H, H), _full_spec(1, H),
            _full_spec(1, H), _full_spec(1, H),
        ],
        out_specs=_row_spec(H),
        out_shape=jax.ShapeDtypeStruct((N, H), jnp.float32),
    )(eps.reshape(1), x, p0, p1, w1, b1.reshape(1, H), g1.reshape(1, H),
      be1.reshape(1, H), w2, b2.reshape(1, H), ng.reshape(1, H), nb.reshape(1, H))


def _mlp2(eps, h, q0, q1, w1, b1, g1, be1, w2, b2, ng, nb,
          rw1, rb1, rg, rbe, rw2, rb2):
    return pl.pallas_call(
        _mlp2_body,
        grid=(NB,),
        in_specs=[
            _smem_spec(), _row_spec(H), _row_spec(H), _row_spec(H),
            _full_spec(H, H), _full_spec(1, H), _full_spec(1, H),
            _full_spec(1, H), _full_spec(H, H), _full_spec(1, H),
            _full_spec(1, H), _full_spec(1, H),
            _full_spec(H, OUT), _full_spec(1, OUT), _full_spec(1, OUT),
            _full_spec(1, OUT), _full_spec(OUT, OUT), _full_spec(1, OUT),
        ],
        out_specs=_row_spec(OUT),
        out_shape=jax.ShapeDtypeStruct((N, OUT), jnp.float32),
    )(eps.reshape(1), h, q0, q1, w1, b1.reshape(1, H), g1.reshape(1, H),
      be1.reshape(1, H), w2, b2.reshape(1, H), ng.reshape(1, H), nb.reshape(1, H),
      rw1, rb1.reshape(1, OUT), rg.reshape(1, OUT), rbe.reshape(1, OUT),
      rw2, rb2.reshape(1, OUT))


def kernel(x, edge_index, edge_weight, eps1, m1_W1, m1_b1, m1_g1, m1_be1,
           m1_W2, m1_b2, n1_g, n1_b, eps2, m2_W1, m2_b1, m2_g1, m2_be1,
           m2_W2, m2_b2, n2_g, n2_b, r_W1, r_b1, r_g, r_be, r_W2, r_b2):
    ei_resh = edge_index.reshape(2, NW, NG, G, CHUNK)
    p = _sc_agg(x, ei_resh)
    h = _mlp1(eps1, x, p[0], p[1], m1_W1, m1_b1, m1_g1, m1_be1,
              m1_W2, m1_b2, n1_g, n1_b)
    q = _sc_agg(h, ei_resh)
    return _mlp2(eps2, h, q[0], q[1], m2_W1, m2_b1, m2_g1, m2_be1,
                 m2_W2, m2_b2, n2_g, n2_b,
                 r_W1, r_b1, r_g, r_be, r_W2, r_b2)
